# baseline (speedup 1.0000x reference)
import functools

import jax
import jax.numpy as jnp
from jax.experimental import pallas as pl
from jax.experimental.pallas import tpu as pltpu

N = 100000
E = 1600000
G = 64

_NODE_BLK = 2000


def _mlp_ln_block(x_ref, agg_ref, w1t_ref, b1_ref, w2t_ref, b2_ref, o_ref):
    h = x_ref[...] + agg_ref[...]
    h = jnp.maximum(jnp.dot(h, w1t_ref[...], preferred_element_type=jnp.float32)
                    + b1_ref[...], 0.0)
    h = jnp.dot(h, w2t_ref[...], preferred_element_type=jnp.float32) + b2_ref[...]
    mu = jnp.mean(h, axis=-1, keepdims=True)
    var = jnp.mean((h - mu) ** 2, axis=-1, keepdims=True)
    h = (h - mu) * jax.lax.rsqrt(var + 1e-5)
    o_ref[...] = jnp.maximum(h, 0.0)


def _mlp_ln(x, agg, W1, b1, W2, b2):
    n, f_in = x.shape
    f_out = W2.shape[0]
    grid = n // _NODE_BLK
    return pl.pallas_call(
        _mlp_ln_block,
        grid=(grid,),
        in_specs=[
            pl.BlockSpec((_NODE_BLK, f_in), lambda i: (i, 0)),
            pl.BlockSpec((_NODE_BLK, f_in), lambda i: (i, 0)),
            pl.BlockSpec(W1.T.shape, lambda i: (0, 0)),
            pl.BlockSpec((1, b1.shape[0]), lambda i: (0, 0)),
            pl.BlockSpec(W2.T.shape, lambda i: (0, 0)),
            pl.BlockSpec((1, b2.shape[0]), lambda i: (0, 0)),
        ],
        out_specs=pl.BlockSpec((_NODE_BLK, f_out), lambda i: (i, 0)),
        out_shape=jax.ShapeDtypeStruct((n, f_out), jnp.float32),
    )(x, agg, W1.T, b1[None, :], W2.T, b2[None, :])


def _edge_stage(h, edge_index, edge_attr, lin_W, lin_b):
    src = edge_index[0]
    dst = edge_index[1]
    e = edge_attr @ lin_W.T + lin_b
    m = jax.nn.relu(jnp.take(h, src, axis=0) + e)
    return jax.ops.segment_sum(m, dst, num_segments=h.shape[0])


def kernel(x, edge_index, edge_attr, batch, lin1_W, lin1_b, mlp1_W1, mlp1_b1,
           mlp1_W2, mlp1_b2, lin2_W, lin2_b, mlp2_W1, mlp2_b1, mlp2_W2,
           mlp2_b2, fc_W1, fc_b1, fc_W2, fc_b2):
    agg1 = _edge_stage(x, edge_index, edge_attr, lin1_W, lin1_b)
    h1 = _mlp_ln(x, agg1, mlp1_W1, mlp1_b1, mlp1_W2, mlp1_b2)
    agg2 = _edge_stage(h1, edge_index, edge_attr, lin2_W, lin2_b)
    h2 = _mlp_ln(h1, agg2, mlp2_W1, mlp2_b1, mlp2_W2, mlp2_b2)

    ones = jnp.ones((h2.shape[0], 1), dtype=h2.dtype)
    counts = jax.ops.segment_sum(ones, batch, num_segments=G)
    mean_p = jax.ops.segment_sum(h2, batch, num_segments=G) / jnp.maximum(counts, 1.0)
    max_p = jax.ops.segment_max(h2, batch, num_segments=G)
    max_p = jnp.where(counts > 0, max_p, 0.0)
    g = jnp.concatenate([mean_p, max_p], axis=1)
    mu = jnp.mean(g, axis=-1, keepdims=True)
    var = jnp.mean((g - mu) ** 2, axis=-1, keepdims=True)
    g = (g - mu) * jax.lax.rsqrt(var + 1e-5)
    out = jax.nn.relu(g @ fc_W1.T + fc_b1) @ fc_W2.T + fc_b2
    return out


# speedup vs baseline: 1.6914x; 1.6914x over previous
import functools

import jax
import jax.numpy as jnp
from jax import lax
from jax.experimental import pallas as pl
from jax.experimental.pallas import tpu as pltpu
from jax.experimental.pallas import tpu_sc as plsc

N = 100000
E = 1600000
G = 64

N_PAD = 102400
ROWS_PER_TILE = N_PAD // 16
CHUNK = 512
SUB = 128
NSUB = CHUNK // SUB
E_PAD = 1605632
N_DUMP = 2048
WB = 400

_MESH = plsc.VectorSubcoreMesh(core_axis_name="c", subcore_axis_name="s")


def _edge_sc_body(L):
    epw = L // 32
    k_chunks = epw // CHUNK

    def body(table_hbm, srcr_hbm, dstr_hbm, e_hbm, zeros_hbm, out_hbm,
             accum_sh, src_v, dst_v, x_v, e_v, sem):
        c = lax.axis_index("c")
        s = lax.axis_index("s")
        w = c * 16 + s
        for j in range(ROWS_PER_TILE // WB):
            pltpu.sync_copy(zeros_hbm.at[pl.ds(0, WB)],
                            accum_sh.at[pl.ds(s * ROWS_PER_TILE + j * WB, WB)])
        plsc.subcore_barrier()

        base_rows = w * (epw // SUB)
        base_e = w * epw

        def chunk(k, carry):
            rb = base_rows + k * NSUB
            eb = base_e + k * CHUNK
            pltpu.sync_copy(srcr_hbm.at[pl.ds(rb, NSUB)], src_v)
            pltpu.sync_copy(dstr_hbm.at[pl.ds(rb, NSUB)], dst_v)
            pltpu.sync_copy(e_hbm.at[pl.ds(eb, CHUNK)], e_v)
            gathers = [
                pltpu.async_copy(table_hbm.at[src_v.at[j]],
                                 x_v.at[pl.ds(j * SUB, SUB)], sem)
                for j in range(NSUB)
            ]
            for g in gathers:
                g.wait()

            def rows(i, carry2):
                r = i * 8
                for u in range(8):
                    x_v[r + u] = jnp.maximum(x_v[r + u] + e_v[r + u], 0.0)
                return carry2

            lax.fori_loop(0, CHUNK // 8, rows, 0)
            for j in range(NSUB):
                pltpu.sync_copy(x_v.at[pl.ds(j * SUB, SUB)],
                                accum_sh.at[dst_v.at[j]], add=True)
            return carry

        lax.fori_loop(0, k_chunks, chunk, 0)
        plsc.subcore_barrier()
        for j in range(ROWS_PER_TILE // WB):
            r0 = s * ROWS_PER_TILE + j * WB
            pltpu.sync_copy(accum_sh.at[pl.ds(r0, WB)], x_v.at[pl.ds(0, WB)])
            pltpu.sync_copy(x_v.at[pl.ds(0, WB)],
                            out_hbm.at[pl.ds(c * N_PAD + r0, WB)])

    return body


def _edge_sc(L, table, srcr, dstr, e, zeros):
    return pl.kernel(
        _edge_sc_body(L),
        out_type=jax.ShapeDtypeStruct((2 * N_PAD, 16), jnp.float32),
        mesh=_MESH,
        scratch_types=[
            pltpu.VMEM_SHARED((N_PAD, 16), jnp.float32),
            pltpu.VMEM((NSUB, SUB), jnp.int32),
            pltpu.VMEM((NSUB, SUB), jnp.int32),
            pltpu.VMEM((CHUNK, 16), jnp.float32),
            pltpu.VMEM((CHUNK, 16), jnp.float32),
            pltpu.SemaphoreType.DMA,
        ],
        compiler_params=pltpu.CompilerParams(use_tc_tiling_on_sc=False),
    )(table, srcr, dstr, e, zeros)


_EBLK = 8192
_NB = E_PAD // _EBLK


def _edge_embed_block(ea_ref, w1_ref, b1_ref, w2_ref, b2_ref, e1_ref, e2_ref):
    ea = ea_ref[...]
    e1_ref[...] = jnp.dot(ea, w1_ref[...], preferred_element_type=jnp.float32) \
        + b1_ref[...]
    e2_ref[...] = jnp.dot(ea, w2_ref[0], preferred_element_type=jnp.float32) \
        + b2_ref[0]


def _edge_embed(ea_pad, lin1_W, lin1_b, lin2_W, lin2_b):
    w1 = jnp.zeros((3, 16), jnp.float32).at[:, :6].set(lin1_W.T)
    b1 = jnp.zeros((1, 16), jnp.float32).at[0, :6].set(lin1_b)
    w2t = lin2_W.T
    w2 = jnp.stack([w2t[:, :16], w2t[:, 16:]])
    b2 = jnp.stack([lin2_b[None, :16], lin2_b[None, 16:]])
    return pl.pallas_call(
        _edge_embed_block,
        grid=(2, _NB),
        in_specs=[
            pl.BlockSpec((_EBLK, 3), lambda i, j: (j, 0)),
            pl.BlockSpec((3, 16), lambda i, j: (0, 0)),
            pl.BlockSpec((1, 16), lambda i, j: (0, 0)),
            pl.BlockSpec((1, 3, 16), lambda i, j: (i, 0, 0)),
            pl.BlockSpec((1, 1, 16), lambda i, j: (i, 0, 0)),
        ],
        out_specs=[
            pl.BlockSpec((_EBLK, 16), lambda i, j: (j, 0)),
            pl.BlockSpec((_EBLK, 16), lambda i, j: (i * _NB + j, 0)),
        ],
        out_shape=[
            jax.ShapeDtypeStruct((E_PAD, 16), jnp.float32),
            jax.ShapeDtypeStruct((2 * E_PAD, 16), jnp.float32),
        ],
    )(ea_pad, w1, b1, w2, b2)


_NODE_BLK = 2000


def _mlp_ln_block(x_ref, agg_ref, w1t_ref, b1_ref, w2t_ref, b2_ref, o_ref):
    h = x_ref[...] + agg_ref[...]
    h = jnp.maximum(jnp.dot(h, w1t_ref[...], preferred_element_type=jnp.float32)
                    + b1_ref[...], 0.0)
    h = jnp.dot(h, w2t_ref[...], preferred_element_type=jnp.float32) + b2_ref[...]
    mu = jnp.mean(h, axis=-1, keepdims=True)
    var = jnp.mean((h - mu) ** 2, axis=-1, keepdims=True)
    h = (h - mu) * jax.lax.rsqrt(var + 1e-5)
    o_ref[...] = jnp.maximum(h, 0.0)


def _mlp_ln(x, agg, W1, b1, W2, b2):
    n, f_in = x.shape
    f_out = W2.shape[0]
    return pl.pallas_call(
        _mlp_ln_block,
        grid=(n // _NODE_BLK,),
        in_specs=[
            pl.BlockSpec((_NODE_BLK, f_in), lambda i: (i, 0)),
            pl.BlockSpec((_NODE_BLK, f_in), lambda i: (i, 0)),
            pl.BlockSpec(W1.T.shape, lambda i: (0, 0)),
            pl.BlockSpec((1, b1.shape[0]), lambda i: (0, 0)),
            pl.BlockSpec(W2.T.shape, lambda i: (0, 0)),
            pl.BlockSpec((1, b2.shape[0]), lambda i: (0, 0)),
        ],
        out_specs=pl.BlockSpec((_NODE_BLK, f_out), lambda i: (i, 0)),
        out_shape=jax.ShapeDtypeStruct((n, f_out), jnp.float32),
    )(x, agg, W1.T, b1[None, :], W2.T, b2[None, :])


def kernel(x, edge_index, edge_attr, batch, lin1_W, lin1_b, mlp1_W1, mlp1_b1,
           mlp1_W2, mlp1_b2, lin2_W, lin2_b, mlp2_W1, mlp2_b1, mlp2_W2,
           mlp2_b2, fc_W1, fc_b1, fc_W2, fc_b2):
    n_extra = E_PAD - E
    src = edge_index[0]
    dst = edge_index[1]
    pad_iota = lax.iota(jnp.int32, n_extra)
    src_pad = jnp.concatenate([src, pad_iota % 50000])
    dst_pad = jnp.concatenate([dst, N + pad_iota % N_DUMP])
    srcr1 = src_pad.reshape(E_PAD // SUB, SUB)
    dstr1 = dst_pad.reshape(E_PAD // SUB, SUB)
    srcr2 = jnp.concatenate([srcr1, srcr1 + N])
    dstr2 = jnp.concatenate([dstr1, dstr1])
    zeros = jnp.zeros((WB, 16), jnp.float32)

    ea_pad = jnp.zeros((E_PAD, 3), jnp.float32).at[:E].set(edge_attr)
    e1, e2 = _edge_embed(ea_pad, lin1_W, lin1_b, lin2_W, lin2_b)

    x16 = jnp.zeros((N, 16), jnp.float32).at[:, :6].set(x)
    out1 = _edge_sc(E_PAD, x16, srcr1, dstr1, e1, zeros)
    agg1 = (out1[:N, :6] + out1[N_PAD:N_PAD + N, :6])
    h1 = _mlp_ln(x, agg1, mlp1_W1, mlp1_b1, mlp1_W2, mlp1_b2)

    table2 = jnp.concatenate([h1[:, :16], h1[:, 16:]])
    out2 = _edge_sc(2 * E_PAD, table2, srcr2, dstr2, e2, zeros)
    agg2 = jnp.concatenate([out2[:N], out2[N_PAD:N_PAD + N]], axis=1)
    h2 = _mlp_ln(h1, agg2, mlp2_W1, mlp2_b1, mlp2_W2, mlp2_b2)

    ones = jnp.ones((h2.shape[0], 1), dtype=h2.dtype)
    counts = jax.ops.segment_sum(ones, batch, num_segments=G)
    mean_p = jax.ops.segment_sum(h2, batch, num_segments=G) / jnp.maximum(counts, 1.0)
    max_p = jax.ops.segment_max(h2, batch, num_segments=G)
    max_p = jnp.where(counts > 0, max_p, 0.0)
    g = jnp.concatenate([mean_p, max_p], axis=1)
    mu = jnp.mean(g, axis=-1, keepdims=True)
    var = jnp.mean((g - mu) ** 2, axis=-1, keepdims=True)
    g = (g - mu) * jax.lax.rsqrt(var + 1e-5)
    out = jax.nn.relu(g @ fc_W1.T + fc_b1) @ fc_W2.T + fc_b2
    return out


# speedup vs baseline: 2.0433x; 1.2080x over previous
import functools

import jax
import jax.numpy as jnp
from jax import lax
from jax.experimental import pallas as pl
from jax.experimental.pallas import tpu as pltpu
from jax.experimental.pallas import tpu_sc as plsc

N = 100000
E = 1600000
G = 64

N_PAD = 102400
ROWS_PER_TILE = N_PAD // 16
CHUNK = 512
SUB = 128
NSUB = CHUNK // SUB
E_PAD = 1605632
N_DUMP = 2048
WB = 400

_MESH = plsc.VectorSubcoreMesh(core_axis_name="c", subcore_axis_name="s")


def _edge_sc_body(L):
    epw = L // 32
    k_chunks = epw // CHUNK

    def body(table_hbm, srcr_hbm, dstr_hbm, e_hbm, out_hbm,
             accum_sh, src_v, dst_v, x_v, e_v, sem):
        c = lax.axis_index("c")
        s = lax.axis_index("s")
        w = c * 16 + s

        def zrows(i, carry2):
            x_v[i] = jnp.zeros((16,), jnp.float32)
            return carry2

        lax.fori_loop(0, WB, zrows, 0)
        for j in range(ROWS_PER_TILE // WB):
            pltpu.sync_copy(x_v.at[pl.ds(0, WB)],
                            accum_sh.at[pl.ds(s * ROWS_PER_TILE + j * WB, WB)])
        plsc.subcore_barrier()

        base_rows = w * (epw // SUB)
        base_ep = w * (epw // 8)

        def chunk(k, carry):
            rb = base_rows + k * NSUB
            ebp = base_ep + k * (CHUNK // 8)
            pltpu.sync_copy(srcr_hbm.at[pl.ds(rb, NSUB)], src_v)
            pltpu.sync_copy(dstr_hbm.at[pl.ds(rb, NSUB)], dst_v)
            pltpu.sync_copy(e_hbm.at[pl.ds(ebp, CHUNK // 8)], e_v)
            gathers = [
                pltpu.async_copy(table_hbm.at[src_v.at[j]],
                                 x_v.at[pl.ds(j * SUB, SUB)], sem)
                for j in range(NSUB)
            ]
            for g in gathers:
                g.wait()

            def rows(i, carry2):
                r = i * 8
                for u in range(8):
                    x_v[r + u] = jnp.maximum(
                        x_v[r + u] + e_v[i, pl.ds(16 * u, 16)], 0.0)
                return carry2

            lax.fori_loop(0, CHUNK // 8, rows, 0)
            for j in range(NSUB):
                pltpu.sync_copy(x_v.at[pl.ds(j * SUB, SUB)],
                                accum_sh.at[dst_v.at[j]], add=True)
            return carry

        lax.fori_loop(0, k_chunks, chunk, 0)
        plsc.subcore_barrier()
        for j in range(ROWS_PER_TILE // WB):
            r0 = s * ROWS_PER_TILE + j * WB
            pltpu.sync_copy(accum_sh.at[pl.ds(r0, WB)], x_v.at[pl.ds(0, WB)])
            pltpu.sync_copy(x_v.at[pl.ds(0, WB)],
                            out_hbm.at[pl.ds(c * N_PAD + r0, WB)])

    return body


def _edge_sc(L, table, srcr, dstr, e):
    return pl.kernel(
        _edge_sc_body(L),
        out_type=jax.ShapeDtypeStruct((2 * N_PAD, 16), jnp.float32),
        mesh=_MESH,
        scratch_types=[
            pltpu.VMEM_SHARED((N_PAD, 16), jnp.float32),
            pltpu.VMEM((NSUB, SUB), jnp.int32),
            pltpu.VMEM((NSUB, SUB), jnp.int32),
            pltpu.VMEM((CHUNK, 16), jnp.float32),
            pltpu.VMEM((CHUNK // 8, 128), jnp.float32),
            pltpu.SemaphoreType.DMA,
        ],
        compiler_params=pltpu.CompilerParams(use_tc_tiling_on_sc=False),
    )(table, srcr, dstr, e)


_EBLK = 8192
_NB = E_PAD // _EBLK
_EBLKP = _EBLK // 8


def _edge_embed_block(ea_ref, w1_ref, b1_ref, w2_ref, b2_ref, e1_ref, e2_ref):
    ea = ea_ref[...]
    e1_ref[...] = jnp.dot(ea, w1_ref[...], preferred_element_type=jnp.float32) \
        + b1_ref[...]
    e2_ref[...] = jnp.dot(ea, w2_ref[0], preferred_element_type=jnp.float32) \
        + b2_ref[0]


def _block_diag8(w):
    out = jnp.zeros((8, 3, 8, 16), jnp.float32)
    out = out.at[jnp.arange(8), :, jnp.arange(8), :].set(
        jnp.broadcast_to(w, (8, 3, 16)))
    return out.reshape(24, 128)


def _edge_embed(ea_packed, lin1_W, lin1_b, lin2_W, lin2_b):
    w1 = _block_diag8(jnp.zeros((3, 16), jnp.float32).at[:, :6].set(lin1_W.T))
    b1 = jnp.tile(jnp.zeros((16,), jnp.float32).at[:6].set(lin1_b), 8)[None, :]
    w2t = lin2_W.T
    w2 = jnp.stack([_block_diag8(w2t[:, :16]), _block_diag8(w2t[:, 16:])])
    b2 = jnp.stack([jnp.tile(lin2_b[:16], 8)[None, :],
                    jnp.tile(lin2_b[16:], 8)[None, :]])
    return pl.pallas_call(
        _edge_embed_block,
        grid=(2, _NB),
        in_specs=[
            pl.BlockSpec((_EBLKP, 24), lambda i, j: (j, 0)),
            pl.BlockSpec((24, 128), lambda i, j: (0, 0)),
            pl.BlockSpec((1, 128), lambda i, j: (0, 0)),
            pl.BlockSpec((1, 24, 128), lambda i, j: (i, 0, 0)),
            pl.BlockSpec((1, 1, 128), lambda i, j: (i, 0, 0)),
        ],
        out_specs=[
            pl.BlockSpec((_EBLKP, 128), lambda i, j: (j, 0)),
            pl.BlockSpec((_EBLKP, 128), lambda i, j: (i * _NB + j, 0)),
        ],
        out_shape=[
            jax.ShapeDtypeStruct((E_PAD // 8, 128), jnp.float32),
            jax.ShapeDtypeStruct((2 * E_PAD // 8, 128), jnp.float32),
        ],
    )(ea_packed, w1, b1, w2, b2)


_NODE_BLK = 2000


def _mlp_ln_block(x_ref, agg_ref, w1t_ref, b1_ref, w2t_ref, b2_ref, o_ref):
    h = x_ref[...] + agg_ref[...]
    h = jnp.maximum(jnp.dot(h, w1t_ref[...], preferred_element_type=jnp.float32)
                    + b1_ref[...], 0.0)
    h = jnp.dot(h, w2t_ref[...], preferred_element_type=jnp.float32) + b2_ref[...]
    mu = jnp.mean(h, axis=-1, keepdims=True)
    var = jnp.mean((h - mu) ** 2, axis=-1, keepdims=True)
    h = (h - mu) * jax.lax.rsqrt(var + 1e-5)
    o_ref[...] = jnp.maximum(h, 0.0)


def _mlp_ln(x, agg, W1, b1, W2, b2):
    n, f_in = x.shape
    f_out = W2.shape[0]
    return pl.pallas_call(
        _mlp_ln_block,
        grid=(n // _NODE_BLK,),
        in_specs=[
            pl.BlockSpec((_NODE_BLK, f_in), lambda i: (i, 0)),
            pl.BlockSpec((_NODE_BLK, f_in), lambda i: (i, 0)),
            pl.BlockSpec(W1.T.shape, lambda i: (0, 0)),
            pl.BlockSpec((1, b1.shape[0]), lambda i: (0, 0)),
            pl.BlockSpec(W2.T.shape, lambda i: (0, 0)),
            pl.BlockSpec((1, b2.shape[0]), lambda i: (0, 0)),
        ],
        out_specs=pl.BlockSpec((_NODE_BLK, f_out), lambda i: (i, 0)),
        out_shape=jax.ShapeDtypeStruct((n, f_out), jnp.float32),
    )(x, agg, W1.T, b1[None, :], W2.T, b2[None, :])


def kernel(x, edge_index, edge_attr, batch, lin1_W, lin1_b, mlp1_W1, mlp1_b1,
           mlp1_W2, mlp1_b2, lin2_W, lin2_b, mlp2_W1, mlp2_b1, mlp2_W2,
           mlp2_b2, fc_W1, fc_b1, fc_W2, fc_b2):
    n_extra = E_PAD - E
    src = edge_index[0]
    dst = edge_index[1]
    pad_iota = lax.iota(jnp.int32, n_extra)
    src_pad = jnp.concatenate([src, pad_iota % 50000])
    dst_pad = jnp.concatenate([dst, N + pad_iota % N_DUMP])
    srcr1 = src_pad.reshape(E_PAD // SUB, SUB)
    dstr1 = dst_pad.reshape(E_PAD // SUB, SUB)
    srcr2 = jnp.concatenate([srcr1, srcr1 + N])
    dstr2 = jnp.concatenate([dstr1, dstr1])

    ea_packed = jnp.zeros((E_PAD, 3), jnp.float32).at[:E].set(edge_attr) \
        .reshape(E_PAD // 8, 24)
    e1, e2 = _edge_embed(ea_packed, lin1_W, lin1_b, lin2_W, lin2_b)

    x16 = jnp.zeros((N, 16), jnp.float32).at[:, :6].set(x)
    out1 = _edge_sc(E_PAD, x16, srcr1, dstr1, e1)
    agg1 = (out1[:N, :6] + out1[N_PAD:N_PAD + N, :6])
    h1 = _mlp_ln(x, agg1, mlp1_W1, mlp1_b1, mlp1_W2, mlp1_b2)

    table2 = jnp.concatenate([h1[:, :16], h1[:, 16:]])
    out2 = _edge_sc(2 * E_PAD, table2, srcr2, dstr2, e2)
    agg2 = jnp.concatenate([out2[:N], out2[N_PAD:N_PAD + N]], axis=1)
    h2 = _mlp_ln(h1, agg2, mlp2_W1, mlp2_b1, mlp2_W2, mlp2_b2)

    ones = jnp.ones((h2.shape[0], 1), dtype=h2.dtype)
    counts = jax.ops.segment_sum(ones, batch, num_segments=G)
    mean_p = jax.ops.segment_sum(h2, batch, num_segments=G) / jnp.maximum(counts, 1.0)
    max_p = jax.ops.segment_max(h2, batch, num_segments=G)
    max_p = jnp.where(counts > 0, max_p, 0.0)
    g = jnp.concatenate([mean_p, max_p], axis=1)
    mu = jnp.mean(g, axis=-1, keepdims=True)
    var = jnp.mean((g - mu) ** 2, axis=-1, keepdims=True)
    g = (g - mu) * jax.lax.rsqrt(var + 1e-5)
    out = jax.nn.relu(g @ fc_W1.T + fc_b1) @ fc_W2.T + fc_b2
    return out


# speedup vs baseline: 2.0506x; 1.0036x over previous
import functools

import jax
import jax.numpy as jnp
from jax import lax
from jax.experimental import pallas as pl
from jax.experimental.pallas import tpu as pltpu
from jax.experimental.pallas import tpu_sc as plsc

N = 100000
E = 1600000
G = 64

N_PAD = 102400
ROWS_PER_TILE = N_PAD // 16
CHUNK = 512
SUB = 128
NSUB = CHUNK // SUB
E_PAD = 1605632
N_DUMP = 2048
WB = 400

_MESH = plsc.VectorSubcoreMesh(core_axis_name="c", subcore_axis_name="s")


def _edge_sc_body(L):
    epw = L // 32
    k_chunks = epw // CHUNK

    def body(table_hbm, srcr_hbm, dstr_hbm, e_hbm, out_hbm,
             accum_sh, src_v, dst_v, x_v, e_v, sem):
        c = lax.axis_index("c")
        s = lax.axis_index("s")
        w = c * 16 + s

        def zrows(i, carry2):
            x_v[i] = jnp.zeros((16,), jnp.float32)
            return carry2

        lax.fori_loop(0, WB, zrows, 0)
        for j in range(ROWS_PER_TILE // WB):
            pltpu.sync_copy(x_v.at[pl.ds(0, WB)],
                            accum_sh.at[pl.ds(s * ROWS_PER_TILE + j * WB, WB)])
        plsc.subcore_barrier()

        base_rows = w * (epw // SUB)
        base_ep = w * (epw // 8)

        def chunk(k, carry):
            rb = base_rows + k * NSUB
            ebp = base_ep + k * (CHUNK // 8)
            pltpu.sync_copy(srcr_hbm.at[pl.ds(rb, NSUB)], src_v)
            pltpu.sync_copy(dstr_hbm.at[pl.ds(rb, NSUB)], dst_v)
            pltpu.sync_copy(e_hbm.at[pl.ds(ebp, CHUNK // 8)], e_v)
            gathers = [
                pltpu.async_copy(table_hbm.at[src_v.at[j]],
                                 x_v.at[pl.ds(j * SUB, SUB)], sem)
                for j in range(NSUB)
            ]
            for g in gathers:
                g.wait()

            def rows(i, carry2):
                r = i * 8
                for u in range(8):
                    x_v[r + u] = jnp.maximum(
                        x_v[r + u] + e_v[i, pl.ds(16 * u, 16)], 0.0)
                return carry2

            lax.fori_loop(0, CHUNK // 8, rows, 0)
            for j in range(NSUB):
                pltpu.sync_copy(x_v.at[pl.ds(j * SUB, SUB)],
                                accum_sh.at[dst_v.at[j]], add=True)
            return carry

        lax.fori_loop(0, k_chunks, chunk, 0)
        plsc.subcore_barrier()
        for j in range(ROWS_PER_TILE // WB):
            r0 = s * ROWS_PER_TILE + j * WB
            pltpu.sync_copy(accum_sh.at[pl.ds(r0, WB)], x_v.at[pl.ds(0, WB)])

            def pk(i, carry2):
                for u in range(8):
                    e_v[i, pl.ds(16 * u, 16)] = x_v[i * 8 + u]
                return carry2

            lax.fori_loop(0, WB // 8, pk, 0)
            pltpu.sync_copy(e_v.at[pl.ds(0, WB // 8)],
                            out_hbm.at[pl.ds((c * N_PAD + r0) // 8, WB // 8)])

    return body


def _edge_sc(L, table, srcr, dstr, e):
    return pl.kernel(
        _edge_sc_body(L),
        out_type=jax.ShapeDtypeStruct((2 * N_PAD // 8, 128), jnp.float32),
        mesh=_MESH,
        scratch_types=[
            pltpu.VMEM_SHARED((N_PAD, 16), jnp.float32),
            pltpu.VMEM((NSUB, SUB), jnp.int32),
            pltpu.VMEM((NSUB, SUB), jnp.int32),
            pltpu.VMEM((CHUNK, 16), jnp.float32),
            pltpu.VMEM((CHUNK // 8, 128), jnp.float32),
            pltpu.SemaphoreType.DMA,
        ],
        compiler_params=pltpu.CompilerParams(use_tc_tiling_on_sc=False),
    )(table, srcr, dstr, e)


def _repack_body(P):
    rows_pt = P // 32
    k_steps = rows_pt // 50

    def body(in_hbm, out_hbm, v_in, v_out):
        c = lax.axis_index("c")
        s = lax.axis_index("s")
        w = c * 16 + s
        base = w * rows_pt

        def go(k, carry):
            pb = base + k * 50
            pltpu.sync_copy(in_hbm.at[pl.ds(pb, 50)], v_in)

            def pk(i, carry2):
                for u in range(8):
                    v_out[i * 8 + u] = v_in[i, pl.ds(16 * u, 16)]
                return carry2

            lax.fori_loop(0, 50, pk, 0)
            pltpu.sync_copy(v_out, out_hbm.at[pl.ds(pb * 8, 400)])
            return carry

        lax.fori_loop(0, k_steps, go, 0)

    return body


def _repack_sc(x_packed):
    P = x_packed.shape[0]
    return pl.kernel(
        _repack_body(P),
        out_type=jax.ShapeDtypeStruct((8 * P, 16), jnp.float32),
        mesh=_MESH,
        scratch_types=[
            pltpu.VMEM((50, 128), jnp.float32),
            pltpu.VMEM((400, 16), jnp.float32),
        ],
        compiler_params=pltpu.CompilerParams(use_tc_tiling_on_sc=False),
    )(x_packed)


_EBLK = 8192
_NB = E_PAD // _EBLK
_EBLKP = _EBLK // 8


def _edge_embed_block(ea_ref, w1_ref, b1_ref, w2_ref, b2_ref, e1_ref, e2_ref):
    ea = ea_ref[...]
    e1_ref[...] = jnp.dot(ea, w1_ref[...], preferred_element_type=jnp.float32) \
        + b1_ref[...]
    e2_ref[...] = jnp.dot(ea, w2_ref[0], preferred_element_type=jnp.float32) \
        + b2_ref[0]


def _block_diag8(w):
    out = jnp.zeros((8, 3, 8, 16), jnp.float32)
    out = out.at[jnp.arange(8), :, jnp.arange(8), :].set(
        jnp.broadcast_to(w, (8, 3, 16)))
    return out.reshape(24, 128)


def _edge_embed(ea_packed, lin1_W, lin1_b, lin2_W, lin2_b):
    w1 = _block_diag8(jnp.zeros((3, 16), jnp.float32).at[:, :6].set(lin1_W.T))
    b1 = jnp.tile(jnp.zeros((16,), jnp.float32).at[:6].set(lin1_b), 8)[None, :]
    w2t = lin2_W.T
    w2 = jnp.stack([_block_diag8(w2t[:, :16]), _block_diag8(w2t[:, 16:])])
    b2 = jnp.stack([jnp.tile(lin2_b[:16], 8)[None, :],
                    jnp.tile(lin2_b[16:], 8)[None, :]])
    return pl.pallas_call(
        _edge_embed_block,
        grid=(2, _NB),
        in_specs=[
            pl.BlockSpec((_EBLKP, 24), lambda i, j: (j, 0)),
            pl.BlockSpec((24, 128), lambda i, j: (0, 0)),
            pl.BlockSpec((1, 128), lambda i, j: (0, 0)),
            pl.BlockSpec((1, 24, 128), lambda i, j: (i, 0, 0)),
            pl.BlockSpec((1, 1, 128), lambda i, j: (i, 0, 0)),
        ],
        out_specs=[
            pl.BlockSpec((_EBLKP, 128), lambda i, j: (j, 0)),
            pl.BlockSpec((_EBLKP, 128), lambda i, j: (i * _NB + j, 0)),
        ],
        out_shape=[
            jax.ShapeDtypeStruct((E_PAD // 8, 128), jnp.float32),
            jax.ShapeDtypeStruct((2 * E_PAD // 8, 128), jnp.float32),
        ],
    )(ea_packed, w1, b1, w2, b2)


_NODE_BLK = 2000


def _mlp_ln_block(x_ref, agg_ref, w1t_ref, b1_ref, w2t_ref, b2_ref, o_ref):
    h = x_ref[...] + agg_ref[...]
    h = jnp.maximum(jnp.dot(h, w1t_ref[...], preferred_element_type=jnp.float32)
                    + b1_ref[...], 0.0)
    h = jnp.dot(h, w2t_ref[...], preferred_element_type=jnp.float32) + b2_ref[...]
    mu = jnp.mean(h, axis=-1, keepdims=True)
    var = jnp.mean((h - mu) ** 2, axis=-1, keepdims=True)
    h = (h - mu) * jax.lax.rsqrt(var + 1e-5)
    o_ref[...] = jnp.maximum(h, 0.0)


def _mlp_ln(x, agg, W1, b1, W2, b2):
    n, f_in = x.shape
    f_out = W2.shape[0]
    return pl.pallas_call(
        _mlp_ln_block,
        grid=(n // _NODE_BLK,),
        in_specs=[
            pl.BlockSpec((_NODE_BLK, f_in), lambda i: (i, 0)),
            pl.BlockSpec((_NODE_BLK, f_in), lambda i: (i, 0)),
            pl.BlockSpec(W1.T.shape, lambda i: (0, 0)),
            pl.BlockSpec((1, b1.shape[0]), lambda i: (0, 0)),
            pl.BlockSpec(W2.T.shape, lambda i: (0, 0)),
            pl.BlockSpec((1, b2.shape[0]), lambda i: (0, 0)),
        ],
        out_specs=pl.BlockSpec((_NODE_BLK, f_out), lambda i: (i, 0)),
        out_shape=jax.ShapeDtypeStruct((n, f_out), jnp.float32),
    )(x, agg, W1.T, b1[None, :], W2.T, b2[None, :])


def kernel(x, edge_index, edge_attr, batch, lin1_W, lin1_b, mlp1_W1, mlp1_b1,
           mlp1_W2, mlp1_b2, lin2_W, lin2_b, mlp2_W1, mlp2_b1, mlp2_W2,
           mlp2_b2, fc_W1, fc_b1, fc_W2, fc_b2):
    n_extra = E_PAD - E
    src = edge_index[0]
    dst = edge_index[1]
    pad_iota = lax.iota(jnp.int32, n_extra)
    src_pad = jnp.concatenate([src, pad_iota % 50000])
    dst_pad = jnp.concatenate([dst, N + pad_iota % N_DUMP])
    srcr1 = src_pad.reshape(E_PAD // SUB, SUB)
    dstr1 = dst_pad.reshape(E_PAD // SUB, SUB)
    srcr2 = jnp.concatenate([srcr1, srcr1 + N_PAD])
    dstr2 = jnp.concatenate([dstr1, dstr1])

    ea_packed = jnp.zeros((E_PAD, 3), jnp.float32).at[:E].set(edge_attr) \
        .reshape(E_PAD // 8, 24)
    e1, e2 = _edge_embed(ea_packed, lin1_W, lin1_b, lin2_W, lin2_b)

    x16p = jnp.zeros((N_PAD, 16), jnp.float32).at[:N, :6].set(x) \
        .reshape(N_PAD // 8, 128)
    table1 = _repack_sc(x16p)
    out1 = _edge_sc(E_PAD, table1, srcr1, dstr1, e1).reshape(2 * N_PAD, 16)
    agg1 = (out1[:N, :6] + out1[N_PAD:N_PAD + N, :6])
    h1 = _mlp_ln(x, agg1, mlp1_W1, mlp1_b1, mlp1_W2, mlp1_b2)

    h1p = jnp.zeros((N_PAD, 32), jnp.float32).at[:N].set(h1)
    t2p = jnp.concatenate([h1p[:, :16].reshape(N_PAD // 8, 128),
                           h1p[:, 16:].reshape(N_PAD // 8, 128)])
    table2 = _repack_sc(t2p)
    out2 = _edge_sc(2 * E_PAD, table2, srcr2, dstr2, e2).reshape(2 * N_PAD, 16)
    agg2 = jnp.concatenate([out2[:N], out2[N_PAD:N_PAD + N]], axis=1)
    h2 = _mlp_ln(h1, agg2, mlp2_W1, mlp2_b1, mlp2_W2, mlp2_b2)

    ones = jnp.ones((h2.shape[0], 1), dtype=h2.dtype)
    counts = jax.ops.segment_sum(ones, batch, num_segments=G)
    mean_p = jax.ops.segment_sum(h2, batch, num_segments=G) / jnp.maximum(counts, 1.0)
    max_p = jax.ops.segment_max(h2, batch, num_segments=G)
    max_p = jnp.where(counts > 0, max_p, 0.0)
    g = jnp.concatenate([mean_p, max_p], axis=1)
    mu = jnp.mean(g, axis=-1, keepdims=True)
    var = jnp.mean((g - mu) ** 2, axis=-1, keepdims=True)
    g = (g - mu) * jax.lax.rsqrt(var + 1e-5)
    out = jax.nn.relu(g @ fc_W1.T + fc_b1) @ fc_W2.T + fc_b2
    return out


# speedup vs baseline: 5.1353x; 2.5043x over previous
import functools

import jax
import jax.numpy as jnp
from jax import lax
from jax.experimental import pallas as pl
from jax.experimental.pallas import tpu as pltpu
from jax.experimental.pallas import tpu_sc as plsc

N = 100000
E = 1600000
G = 64

N_PAD = 102400
ROWS_PER_TILE = N_PAD // 16
CHUNK = 512
SUB = 128
NSUB = CHUNK // SUB
E_PAD = 1605632
N_DUMP = 2048
WB = 400

_MESH = plsc.VectorSubcoreMesh(core_axis_name="c", subcore_axis_name="s")


def _edge_sc_body(L):
    epw = L // 32
    k_chunks = epw // CHUNK

    def body(table_hbm, srcr_hbm, dstr_hbm, a0_hbm, a1_hbm, a2_hbm, wts_hbm,
             out_hbm, accum_sh, src_v, dst_v, x_v, e_v, a0_v, a1_v, a2_v,
             w_v, sem):
        c = lax.axis_index("c")
        s = lax.axis_index("s")
        w = c * 16 + s
        pltpu.sync_copy(wts_hbm, w_v)

        def zrows(i, carry2):
            x_v[i] = jnp.zeros((16,), jnp.float32)
            return carry2

        lax.fori_loop(0, WB, zrows, 0)
        for j in range(ROWS_PER_TILE // WB):
            pltpu.sync_copy(x_v.at[pl.ds(0, WB)],
                            accum_sh.at[pl.ds(s * ROWS_PER_TILE + j * WB, WB)])
        plsc.subcore_barrier()

        base_rows = w * (epw // SUB)
        base_e = w * epw
        w0 = w_v[c, pl.ds(0, 16)]
        w1 = w_v[c, pl.ds(16, 16)]
        w2 = w_v[c, pl.ds(32, 16)]
        wb = w_v[c, pl.ds(48, 16)]

        def chunk(k, carry):
            rb = base_rows + k * NSUB
            eb = base_e + k * CHUNK
            pltpu.sync_copy(srcr_hbm.at[pl.ds(rb, NSUB)], src_v)
            pltpu.sync_copy(dstr_hbm.at[pl.ds(rb, NSUB)], dst_v)
            pltpu.sync_copy(a0_hbm.at[pl.ds(eb, CHUNK)], a0_v)
            pltpu.sync_copy(a1_hbm.at[pl.ds(eb, CHUNK)], a1_v)
            pltpu.sync_copy(a2_hbm.at[pl.ds(eb, CHUNK)], a2_v)
            gathers = [
                pltpu.async_copy(table_hbm.at[src_v.at[j]],
                                 x_v.at[pl.ds(j * SUB, SUB)], sem)
                for j in range(NSUB)
            ]
            for g in gathers:
                g.wait()

            def rows(i, carry2):
                r = i * 16
                a0g = a0_v[pl.ds(r, 16)]
                a1g = a1_v[pl.ds(r, 16)]
                a2g = a2_v[pl.ds(r, 16)]
                for u in range(16):
                    e = wb + a0g[u] * w0 + a1g[u] * w1 + a2g[u] * w2
                    x_v[r + u] = jnp.maximum(x_v[r + u] + e, 0.0)
                return carry2

            lax.fori_loop(0, CHUNK // 16, rows, 0)
            for j in range(NSUB):
                pltpu.sync_copy(x_v.at[pl.ds(j * SUB, SUB)],
                                accum_sh.at[dst_v.at[j]], add=True)
            return carry

        lax.fori_loop(0, k_chunks, chunk, 0)
        plsc.subcore_barrier()
        for j in range(ROWS_PER_TILE // WB):
            r0 = s * ROWS_PER_TILE + j * WB
            pltpu.sync_copy(accum_sh.at[pl.ds(r0, WB)], x_v.at[pl.ds(0, WB)])

            def pk(i, carry2):
                for u in range(8):
                    e_v[i, pl.ds(16 * u, 16)] = x_v[i * 8 + u]
                return carry2

            lax.fori_loop(0, WB // 8, pk, 0)
            pltpu.sync_copy(e_v.at[pl.ds(0, WB // 8)],
                            out_hbm.at[pl.ds((c * N_PAD + r0) // 8, WB // 8)])

    return body


def _edge_sc(L, table, srcr, dstr, a0, a1, a2, wts):
    return pl.kernel(
        _edge_sc_body(L),
        out_type=jax.ShapeDtypeStruct((2 * N_PAD // 8, 128), jnp.float32),
        mesh=_MESH,
        scratch_types=[
            pltpu.VMEM_SHARED((N_PAD, 16), jnp.float32),
            pltpu.VMEM((NSUB, SUB), jnp.int32),
            pltpu.VMEM((NSUB, SUB), jnp.int32),
            pltpu.VMEM((CHUNK, 16), jnp.float32),
            pltpu.VMEM((CHUNK // 8, 128), jnp.float32),
            pltpu.VMEM((CHUNK,), jnp.float32),
            pltpu.VMEM((CHUNK,), jnp.float32),
            pltpu.VMEM((CHUNK,), jnp.float32),
            pltpu.VMEM((2, 128), jnp.float32),
            pltpu.SemaphoreType.DMA,
        ],
        compiler_params=pltpu.CompilerParams(use_tc_tiling_on_sc=False),
    )(table, srcr, dstr, a0, a1, a2, wts)


def _repack_body(P):
    rows_pt = P // 32
    k_steps = rows_pt // 50

    def body(in_hbm, out_hbm, v_in, v_out):
        c = lax.axis_index("c")
        s = lax.axis_index("s")
        w = c * 16 + s
        base = w * rows_pt

        def go(k, carry):
            pb = base + k * 50
            pltpu.sync_copy(in_hbm.at[pl.ds(pb, 50)], v_in)

            def pk(i, carry2):
                for u in range(8):
                    v_out[i * 8 + u] = v_in[i, pl.ds(16 * u, 16)]
                return carry2

            lax.fori_loop(0, 50, pk, 0)
            pltpu.sync_copy(v_out, out_hbm.at[pl.ds(pb * 8, 400)])
            return carry

        lax.fori_loop(0, k_steps, go, 0)

    return body


def _repack_sc(x_packed):
    P = x_packed.shape[0]
    return pl.kernel(
        _repack_body(P),
        out_type=jax.ShapeDtypeStruct((8 * P, 16), jnp.float32),
        mesh=_MESH,
        scratch_types=[
            pltpu.VMEM((50, 128), jnp.float32),
            pltpu.VMEM((400, 16), jnp.float32),
        ],
        compiler_params=pltpu.CompilerParams(use_tc_tiling_on_sc=False),
    )(x_packed)


def _wts_row(w3x16, b16):
    return jnp.concatenate([w3x16[0], w3x16[1], w3x16[2], b16,
                            jnp.zeros((64,), jnp.float32)])


_NODE_BLK = 2000


def _mlp_ln_block(x_ref, agg_ref, w1t_ref, b1_ref, w2t_ref, b2_ref, o_ref):
    h = x_ref[...] + agg_ref[...]
    h = jnp.maximum(jnp.dot(h, w1t_ref[...], preferred_element_type=jnp.float32)
                    + b1_ref[...], 0.0)
    h = jnp.dot(h, w2t_ref[...], preferred_element_type=jnp.float32) + b2_ref[...]
    mu = jnp.mean(h, axis=-1, keepdims=True)
    var = jnp.mean((h - mu) ** 2, axis=-1, keepdims=True)
    h = (h - mu) * jax.lax.rsqrt(var + 1e-5)
    o_ref[...] = jnp.maximum(h, 0.0)


def _mlp_ln(x, agg, W1, b1, W2, b2):
    n, f_in = x.shape
    f_out = W2.shape[0]
    return pl.pallas_call(
        _mlp_ln_block,
        grid=(n // _NODE_BLK,),
        in_specs=[
            pl.BlockSpec((_NODE_BLK, f_in), lambda i: (i, 0)),
            pl.BlockSpec((_NODE_BLK, f_in), lambda i: (i, 0)),
            pl.BlockSpec(W1.T.shape, lambda i: (0, 0)),
            pl.BlockSpec((1, b1.shape[0]), lambda i: (0, 0)),
            pl.BlockSpec(W2.T.shape, lambda i: (0, 0)),
            pl.BlockSpec((1, b2.shape[0]), lambda i: (0, 0)),
        ],
        out_specs=pl.BlockSpec((_NODE_BLK, f_out), lambda i: (i, 0)),
        out_shape=jax.ShapeDtypeStruct((n, f_out), jnp.float32),
    )(x, agg, W1.T, b1[None, :], W2.T, b2[None, :])


def kernel(x, edge_index, edge_attr, batch, lin1_W, lin1_b, mlp1_W1, mlp1_b1,
           mlp1_W2, mlp1_b2, lin2_W, lin2_b, mlp2_W1, mlp2_b1, mlp2_W2,
           mlp2_b2, fc_W1, fc_b1, fc_W2, fc_b2):
    n_extra = E_PAD - E
    src = edge_index[0]
    dst = edge_index[1]
    pad_iota = lax.iota(jnp.int32, n_extra)
    src_pad = jnp.concatenate([src, pad_iota % 50000])
    dst_pad = jnp.concatenate([dst, N + pad_iota % N_DUMP])
    srcr1 = src_pad.reshape(E_PAD // SUB, SUB)
    dstr1 = dst_pad.reshape(E_PAD // SUB, SUB)
    srcr2 = jnp.concatenate([srcr1, srcr1 + N_PAD])
    dstr2 = jnp.concatenate([dstr1, dstr1])

    a0 = jnp.zeros((E_PAD,), jnp.float32).at[:E].set(edge_attr[:, 0])
    a1 = jnp.zeros((E_PAD,), jnp.float32).at[:E].set(edge_attr[:, 1])
    a2 = jnp.zeros((E_PAD,), jnp.float32).at[:E].set(edge_attr[:, 2])
    w1t = jnp.zeros((3, 16), jnp.float32).at[:, :6].set(lin1_W.T)
    b1p = jnp.zeros((16,), jnp.float32).at[:6].set(lin1_b)
    wts1 = jnp.stack([_wts_row(w1t, b1p)] * 2)
    w2t = lin2_W.T
    wts2 = jnp.stack([_wts_row(w2t[:, :16], lin2_b[:16]),
                      _wts_row(w2t[:, 16:], lin2_b[16:])])

    x16p = jnp.zeros((N_PAD, 16), jnp.float32).at[:N, :6].set(x) \
        .reshape(N_PAD // 8, 128)
    table1 = _repack_sc(x16p)
    out1 = _edge_sc(E_PAD, table1, srcr1, dstr1, a0, a1, a2,
                    wts1).reshape(2 * N_PAD, 16)
    agg1 = (out1[:N, :6] + out1[N_PAD:N_PAD + N, :6])
    h1 = _mlp_ln(x, agg1, mlp1_W1, mlp1_b1, mlp1_W2, mlp1_b2)

    h1p = jnp.zeros((N_PAD, 32), jnp.float32).at[:N].set(h1)
    t2p = jnp.concatenate([h1p[:, :16].reshape(N_PAD // 8, 128),
                           h1p[:, 16:].reshape(N_PAD // 8, 128)])
    table2 = _repack_sc(t2p)
    a0d = jnp.concatenate([a0, a0])
    a1d = jnp.concatenate([a1, a1])
    a2d = jnp.concatenate([a2, a2])
    out2 = _edge_sc(2 * E_PAD, table2, srcr2, dstr2, a0d, a1d, a2d,
                    wts2).reshape(2 * N_PAD, 16)
    agg2 = jnp.concatenate([out2[:N], out2[N_PAD:N_PAD + N]], axis=1)
    h2 = _mlp_ln(h1, agg2, mlp2_W1, mlp2_b1, mlp2_W2, mlp2_b2)

    ones = jnp.ones((h2.shape[0], 1), dtype=h2.dtype)
    counts = jax.ops.segment_sum(ones, batch, num_segments=G)
    mean_p = jax.ops.segment_sum(h2, batch, num_segments=G) / jnp.maximum(counts, 1.0)
    max_p = jax.ops.segment_max(h2, batch, num_segments=G)
    max_p = jnp.where(counts > 0, max_p, 0.0)
    g = jnp.concatenate([mean_p, max_p], axis=1)
    mu = jnp.mean(g, axis=-1, keepdims=True)
    var = jnp.mean((g - mu) ** 2, axis=-1, keepdims=True)
    g = (g - mu) * jax.lax.rsqrt(var + 1e-5)
    out = jax.nn.relu(g @ fc_W1.T + fc_b1) @ fc_W2.T + fc_b2
    return out


# speedup vs baseline: 5.3216x; 1.0363x over previous
import functools

import jax
import jax.numpy as jnp
from jax import lax
from jax.experimental import pallas as pl
from jax.experimental.pallas import tpu as pltpu
from jax.experimental.pallas import tpu_sc as plsc

N = 100000
E = 1600000
G = 64

N_PAD = 102400
ROWS_PER_TILE = N_PAD // 16
CHUNK = 512
SUB = 128
NSUB = CHUNK // SUB
E_PAD = 1605632
N_DUMP = 2048
WB = 400

_MESH = plsc.VectorSubcoreMesh(core_axis_name="c", subcore_axis_name="s")


def _edge_sc_body(L):
    epw = L // 32
    k_chunks = epw // CHUNK

    def body(table_hbm, srcr_hbm, dstr_hbm, a0_hbm, a1_hbm, a2_hbm, wts_hbm,
             out_hbm, accum_sh, src_v, dst_v, x_v, e_v, a0_v, a1_v, a2_v,
             w_v, sem):
        c = lax.axis_index("c")
        s = lax.axis_index("s")
        w = c * 16 + s
        pltpu.sync_copy(wts_hbm, w_v)

        @plsc.parallel_loop(0, WB, unroll=8)
        def zrows(i):
            x_v[i] = jnp.zeros((16,), jnp.float32)
        for j in range(ROWS_PER_TILE // WB):
            pltpu.sync_copy(x_v.at[pl.ds(0, WB)],
                            accum_sh.at[pl.ds(s * ROWS_PER_TILE + j * WB, WB)])
        plsc.subcore_barrier()

        base_rows = w * (epw // SUB)
        base_e = w * epw
        w0 = w_v[c, pl.ds(0, 16)]
        w1 = w_v[c, pl.ds(16, 16)]
        w2 = w_v[c, pl.ds(32, 16)]
        wb = w_v[c, pl.ds(48, 16)]

        def chunk(k, carry):
            rb = base_rows + k * NSUB
            eb = base_e + k * CHUNK
            pltpu.sync_copy(srcr_hbm.at[pl.ds(rb, NSUB)], src_v)
            pltpu.sync_copy(dstr_hbm.at[pl.ds(rb, NSUB)], dst_v)
            pltpu.sync_copy(a0_hbm.at[pl.ds(eb, CHUNK)], a0_v)
            pltpu.sync_copy(a1_hbm.at[pl.ds(eb, CHUNK)], a1_v)
            pltpu.sync_copy(a2_hbm.at[pl.ds(eb, CHUNK)], a2_v)
            gathers = [
                pltpu.async_copy(table_hbm.at[src_v.at[j]],
                                 x_v.at[pl.ds(j * SUB, SUB)], sem)
                for j in range(NSUB)
            ]
            for g in gathers:
                g.wait()

            @plsc.parallel_loop(0, CHUNK // 16, unroll=2)
            def rows(i):
                r = i * 16
                a0g = a0_v[pl.ds(r, 16)]
                a1g = a1_v[pl.ds(r, 16)]
                a2g = a2_v[pl.ds(r, 16)]
                for u in range(16):
                    e = wb + a0g[u] * w0 + a1g[u] * w1 + a2g[u] * w2
                    x_v[r + u] = jnp.maximum(x_v[r + u] + e, 0.0)
            for j in range(NSUB):
                pltpu.sync_copy(x_v.at[pl.ds(j * SUB, SUB)],
                                accum_sh.at[dst_v.at[j]], add=True)
            return carry

        lax.fori_loop(0, k_chunks, chunk, 0)
        plsc.subcore_barrier()
        for j in range(ROWS_PER_TILE // WB):
            r0 = s * ROWS_PER_TILE + j * WB
            pltpu.sync_copy(accum_sh.at[pl.ds(r0, WB)], x_v.at[pl.ds(0, WB)])

            @plsc.parallel_loop(0, WB // 8, unroll=4)
            def pk(i):
                for u in range(8):
                    e_v[i, pl.ds(16 * u, 16)] = x_v[i * 8 + u]
            pltpu.sync_copy(e_v.at[pl.ds(0, WB // 8)],
                            out_hbm.at[pl.ds((c * N_PAD + r0) // 8, WB // 8)])

    return body


def _edge_sc(L, table, srcr, dstr, a0, a1, a2, wts):
    return pl.kernel(
        _edge_sc_body(L),
        out_type=jax.ShapeDtypeStruct((2 * N_PAD // 8, 128), jnp.float32),
        mesh=_MESH,
        scratch_types=[
            pltpu.VMEM_SHARED((N_PAD, 16), jnp.float32),
            pltpu.VMEM((NSUB, SUB), jnp.int32),
            pltpu.VMEM((NSUB, SUB), jnp.int32),
            pltpu.VMEM((CHUNK, 16), jnp.float32),
            pltpu.VMEM((CHUNK // 8, 128), jnp.float32),
            pltpu.VMEM((CHUNK,), jnp.float32),
            pltpu.VMEM((CHUNK,), jnp.float32),
            pltpu.VMEM((CHUNK,), jnp.float32),
            pltpu.VMEM((2, 128), jnp.float32),
            pltpu.SemaphoreType.DMA,
        ],
        compiler_params=pltpu.CompilerParams(use_tc_tiling_on_sc=False),
    )(table, srcr, dstr, a0, a1, a2, wts)


def _repack_body(P):
    rows_pt = P // 32
    k_steps = rows_pt // 50

    def body(in_hbm, out_hbm, v_in, v_out):
        c = lax.axis_index("c")
        s = lax.axis_index("s")
        w = c * 16 + s
        base = w * rows_pt

        def go(k, carry):
            pb = base + k * 50
            pltpu.sync_copy(in_hbm.at[pl.ds(pb, 50)], v_in)

            @plsc.parallel_loop(0, 50, unroll=4)
            def pk(i):
                for u in range(8):
                    v_out[i * 8 + u] = v_in[i, pl.ds(16 * u, 16)]
            pltpu.sync_copy(v_out, out_hbm.at[pl.ds(pb * 8, 400)])
            return carry

        lax.fori_loop(0, k_steps, go, 0)

    return body


def _repack_sc(x_packed):
    P = x_packed.shape[0]
    return pl.kernel(
        _repack_body(P),
        out_type=jax.ShapeDtypeStruct((8 * P, 16), jnp.float32),
        mesh=_MESH,
        scratch_types=[
            pltpu.VMEM((50, 128), jnp.float32),
            pltpu.VMEM((400, 16), jnp.float32),
        ],
        compiler_params=pltpu.CompilerParams(use_tc_tiling_on_sc=False),
    )(x_packed)


def _wts_row(w3x16, b16):
    return jnp.concatenate([w3x16[0], w3x16[1], w3x16[2], b16,
                            jnp.zeros((64,), jnp.float32)])


_NODE_BLK = 2000


def _mlp_ln_block(x_ref, agg_ref, w1t_ref, b1_ref, w2t_ref, b2_ref, o_ref):
    h = x_ref[...] + agg_ref[...]
    h = jnp.maximum(jnp.dot(h, w1t_ref[...], preferred_element_type=jnp.float32)
                    + b1_ref[...], 0.0)
    h = jnp.dot(h, w2t_ref[...], preferred_element_type=jnp.float32) + b2_ref[...]
    mu = jnp.mean(h, axis=-1, keepdims=True)
    var = jnp.mean((h - mu) ** 2, axis=-1, keepdims=True)
    h = (h - mu) * jax.lax.rsqrt(var + 1e-5)
    o_ref[...] = jnp.maximum(h, 0.0)


def _mlp_ln(x, agg, W1, b1, W2, b2):
    n, f_in = x.shape
    f_out = W2.shape[0]
    return pl.pallas_call(
        _mlp_ln_block,
        grid=(n // _NODE_BLK,),
        in_specs=[
            pl.BlockSpec((_NODE_BLK, f_in), lambda i: (i, 0)),
            pl.BlockSpec((_NODE_BLK, f_in), lambda i: (i, 0)),
            pl.BlockSpec(W1.T.shape, lambda i: (0, 0)),
            pl.BlockSpec((1, b1.shape[0]), lambda i: (0, 0)),
            pl.BlockSpec(W2.T.shape, lambda i: (0, 0)),
            pl.BlockSpec((1, b2.shape[0]), lambda i: (0, 0)),
        ],
        out_specs=pl.BlockSpec((_NODE_BLK, f_out), lambda i: (i, 0)),
        out_shape=jax.ShapeDtypeStruct((n, f_out), jnp.float32),
    )(x, agg, W1.T, b1[None, :], W2.T, b2[None, :])


def kernel(x, edge_index, edge_attr, batch, lin1_W, lin1_b, mlp1_W1, mlp1_b1,
           mlp1_W2, mlp1_b2, lin2_W, lin2_b, mlp2_W1, mlp2_b1, mlp2_W2,
           mlp2_b2, fc_W1, fc_b1, fc_W2, fc_b2):
    n_extra = E_PAD - E
    src = edge_index[0]
    dst = edge_index[1]
    pad_iota = lax.iota(jnp.int32, n_extra)
    src_pad = jnp.concatenate([src, pad_iota % 50000])
    dst_pad = jnp.concatenate([dst, N + pad_iota % N_DUMP])
    srcr1 = src_pad.reshape(E_PAD // SUB, SUB)
    dstr1 = dst_pad.reshape(E_PAD // SUB, SUB)
    srcr2 = jnp.concatenate([srcr1, srcr1 + N_PAD])
    dstr2 = jnp.concatenate([dstr1, dstr1])

    a0 = jnp.zeros((E_PAD,), jnp.float32).at[:E].set(edge_attr[:, 0])
    a1 = jnp.zeros((E_PAD,), jnp.float32).at[:E].set(edge_attr[:, 1])
    a2 = jnp.zeros((E_PAD,), jnp.float32).at[:E].set(edge_attr[:, 2])
    w1t = jnp.zeros((3, 16), jnp.float32).at[:, :6].set(lin1_W.T)
    b1p = jnp.zeros((16,), jnp.float32).at[:6].set(lin1_b)
    wts1 = jnp.stack([_wts_row(w1t, b1p)] * 2)
    w2t = lin2_W.T
    wts2 = jnp.stack([_wts_row(w2t[:, :16], lin2_b[:16]),
                      _wts_row(w2t[:, 16:], lin2_b[16:])])

    x16p = jnp.zeros((N_PAD, 16), jnp.float32).at[:N, :6].set(x) \
        .reshape(N_PAD // 8, 128)
    table1 = _repack_sc(x16p)
    out1 = _edge_sc(E_PAD, table1, srcr1, dstr1, a0, a1, a2,
                    wts1).reshape(2 * N_PAD, 16)
    agg1 = (out1[:N, :6] + out1[N_PAD:N_PAD + N, :6])
    h1 = _mlp_ln(x, agg1, mlp1_W1, mlp1_b1, mlp1_W2, mlp1_b2)

    h1p = jnp.zeros((N_PAD, 32), jnp.float32).at[:N].set(h1)
    t2p = jnp.concatenate([h1p[:, :16].reshape(N_PAD // 8, 128),
                           h1p[:, 16:].reshape(N_PAD // 8, 128)])
    table2 = _repack_sc(t2p)
    a0d = jnp.concatenate([a0, a0])
    a1d = jnp.concatenate([a1, a1])
    a2d = jnp.concatenate([a2, a2])
    out2 = _edge_sc(2 * E_PAD, table2, srcr2, dstr2, a0d, a1d, a2d,
                    wts2).reshape(2 * N_PAD, 16)
    agg2 = jnp.concatenate([out2[:N], out2[N_PAD:N_PAD + N]], axis=1)
    h2 = _mlp_ln(h1, agg2, mlp2_W1, mlp2_b1, mlp2_W2, mlp2_b2)

    ones = jnp.ones((h2.shape[0], 1), dtype=h2.dtype)
    counts = jax.ops.segment_sum(ones, batch, num_segments=G)
    mean_p = jax.ops.segment_sum(h2, batch, num_segments=G) / jnp.maximum(counts, 1.0)
    max_p = jax.ops.segment_max(h2, batch, num_segments=G)
    max_p = jnp.where(counts > 0, max_p, 0.0)
    g = jnp.concatenate([mean_p, max_p], axis=1)
    mu = jnp.mean(g, axis=-1, keepdims=True)
    var = jnp.mean((g - mu) ** 2, axis=-1, keepdims=True)
    g = (g - mu) * jax.lax.rsqrt(var + 1e-5)
    out = jax.nn.relu(g @ fc_W1.T + fc_b1) @ fc_W2.T + fc_b2
    return out
